# baseline (speedup 1.0000x reference)
import functools

import jax
import jax.numpy as jnp
import numpy as np
from jax import lax
from jax.experimental import pallas as pl
from jax.experimental.pallas import tpu as pltpu
from jax.experimental.pallas import tpu_sc as plsc

_LANES = 16


def _threefry2x32_np(k1, k2, x1, x2):
    rot = [(13, 15, 26, 6), (17, 29, 16, 24)]
    ks = [np.uint32(k1), np.uint32(k2),
          np.uint32(np.uint32(k1) ^ np.uint32(k2) ^ np.uint32(0x1BD11BDA))]
    x0 = (x1 + ks[0]).astype(np.uint32)
    x1v = (x2 + ks[1]).astype(np.uint32)
    kseq = [ks[1], ks[2], ks[0]]
    rots = [rot[0], rot[1]]
    for i in range(5):
        for r in rots[0]:
            x0 = (x0 + x1v).astype(np.uint32)
            x1v = ((x1v << np.uint32(r)) | (x1v >> np.uint32(32 - r))).astype(np.uint32)
            x1v = x0 ^ x1v
        x0 = (x0 + kseq[0]).astype(np.uint32)
        x1v = (x1v + kseq[1] + np.uint32(i + 1)).astype(np.uint32)
        kseq = kseq[1:] + kseq[:1]
        rots = rots[1:] + rots[:1]
    return x0, x1v


def _random_bits_np(k, n, partitionable):
    if partitionable:
        b1, b2 = _threefry2x32_np(
            k[0], k[1], np.zeros(n, np.uint32), np.arange(n, dtype=np.uint32))
        return b1 ^ b2
    cnt = np.arange(n, dtype=np.uint32)
    if n % 2:
        cnt = np.append(cnt, np.uint32(0))
    h = len(cnt) // 2
    b1, b2 = _threefry2x32_np(k[0], k[1], cnt[:h], cnt[h:])
    return np.concatenate([b1, b2])[:n]


def _split_np(k1, k2, partitionable):
    if partitionable:
        b1, b2 = _threefry2x32_np(
            k1, k2, np.zeros(2, np.uint32), np.arange(2, dtype=np.uint32))
        return [(b1[0], b2[0]), (b1[1], b2[1])]
    b1, b2 = _threefry2x32_np(
        k1, k2, np.array([0, 1], np.uint32), np.array([2, 3], np.uint32))
    o = np.concatenate([b1, b2])
    return [(o[0], o[1]), (o[2], o[3])]


@functools.lru_cache(maxsize=None)
def _comp_ind_np(batch_size: int, num_samples: int, num_comp: int):
    partitionable = bool(jax.config.jax_threefry_partitionable)
    n = batch_size * num_samples
    khi, klo = _split_np(np.uint32(0), np.uint32(42), partitionable)
    hi_bits = _random_bits_np(khi, n, partitionable)
    lo_bits = _random_bits_np(klo, n, partitionable)
    span = np.uint32(num_comp)
    mult = np.uint32(((2 ** 16 % num_comp) ** 2 % 2 ** 32) % num_comp)
    off = ((hi_bits % span) * mult + (lo_bits % span)).astype(np.uint32) % span
    return off.astype(np.int32)


@functools.lru_cache(maxsize=None)
def _build_sc_kernel(batch: int, nsamp: int, d: int, num_comp: int):
    info = plsc.get_sparse_core_info()
    nc, ns = info.num_cores, info.num_subcores
    nw = nc * ns
    assert d % nw == 0 and batch % _LANES == 0 and num_comp % 8 == 0
    ch_pw = d // nw

    mesh = plsc.VectorSubcoreMesh(core_axis_name="c", subcore_axis_name="s")

    @functools.partial(
        pl.kernel,
        out_type=jax.ShapeDtypeStruct((nsamp, d, batch), jnp.float32),
        mesh=mesh,
        compiler_params=pltpu.CompilerParams(
            use_tc_tiling_on_sc=False, needs_layout_passes=False),
        scratch_types=[
            pltpu.VMEM((num_comp + (-num_comp) % 128,), jnp.float32),
            pltpu.VMEM((2, batch), jnp.int32),
            pltpu.VMEM((2, batch), jnp.float32),
            pltpu.SemaphoreType.DMA((2,)),
            pltpu.SemaphoreType.DMA((2,)),
            pltpu.SemaphoreType.DMA((2,)),
        ],
    )
    def gmm(idx_hbm, noise_hbm, means_hbm, out_hbm,
            table_v, idx_v, noise_v, idx_sem, noi_sem, out_sem):
        wid = lax.axis_index("s") * nc + lax.axis_index("c")

        def idx_copy(s, sl):
            return pltpu.make_async_copy(
                idx_hbm.at[pl.ds(s * batch, batch)], idx_v.at[sl], idx_sem.at[sl])

        def noi_copy(s, c, sl):
            return pltpu.make_async_copy(
                noise_hbm.at[s, c], noise_v.at[sl], noi_sem.at[sl])

        def out_copy(s, c, sl):
            return pltpu.make_async_copy(
                noise_v.at[sl], out_hbm.at[s, c], out_sem.at[sl])

        for ci in range(ch_pw):
            c = wid + nw * ci
            pltpu.sync_copy(means_hbm.at[c], table_v)
            idx_copy(0, 0).start()
            noi_copy(0, c, 0).start()

            def sample(s, carry):
                sl = lax.rem(s, 2)
                s1 = lax.rem(s + 1, 2)

                @pl.when(s + 1 < nsamp)
                def _():
                    @pl.when(s >= 1)
                    def _():
                        out_copy(s - 1, c, s1).wait()
                    idx_copy(s + 1, s1).start()
                    noi_copy(s + 1, c, s1).start()

                idx_copy(s, sl).wait()
                noi_copy(s, c, sl).wait()

                @plsc.parallel_loop(0, batch, step=_LANES, unroll=8)
                def _(i):
                    v = pl.ds(i, _LANES)
                    iv = idx_v[sl, v]
                    g = plsc.load_gather(table_v, [iv])
                    noise_v[sl, v] = noise_v[sl, v] * 0.1 + g
                out_copy(s, c, sl).start()
                return carry

            lax.fori_loop(0, nsamp, sample, 0)
            out_copy(nsamp - 1, c, (nsamp - 1) % 2).wait()
            out_copy(nsamp - 2, c, (nsamp - 2) % 2).wait()

    return gmm


def kernel(input, noise, target_size, means):
    del input, target_size
    b, s, d = noise.shape
    num_comp = means.shape[0]
    idx_np = _comp_ind_np(b, s, num_comp)
    idx_t = jnp.asarray(np.ascontiguousarray(idx_np.reshape(b, s).T).reshape(-1))
    noise_t = jnp.transpose(noise, (1, 2, 0))
    means_t = jnp.pad(jnp.transpose(means, (1, 0)),
                      ((0, 0), (0, (-num_comp) % 128)))
    out_t = _build_sc_kernel(b, s, d, num_comp)(idx_t, noise_t, means_t)
    return jnp.transpose(out_t, (2, 0, 1))


# speedup vs baseline: 1.5357x; 1.5357x over previous
import functools

import jax
import jax.numpy as jnp
import numpy as np
from jax import lax
from jax.experimental import pallas as pl
from jax.experimental.pallas import tpu as pltpu
from jax.experimental.pallas import tpu_sc as plsc

_LANES = 16


def _threefry2x32_np(k1, k2, x1, x2):
    rot = [(13, 15, 26, 6), (17, 29, 16, 24)]
    ks = [np.uint32(k1), np.uint32(k2),
          np.uint32(np.uint32(k1) ^ np.uint32(k2) ^ np.uint32(0x1BD11BDA))]
    x0 = (x1 + ks[0]).astype(np.uint32)
    x1v = (x2 + ks[1]).astype(np.uint32)
    kseq = [ks[1], ks[2], ks[0]]
    rots = [rot[0], rot[1]]
    for i in range(5):
        for r in rots[0]:
            x0 = (x0 + x1v).astype(np.uint32)
            x1v = ((x1v << np.uint32(r)) | (x1v >> np.uint32(32 - r))).astype(np.uint32)
            x1v = x0 ^ x1v
        x0 = (x0 + kseq[0]).astype(np.uint32)
        x1v = (x1v + kseq[1] + np.uint32(i + 1)).astype(np.uint32)
        kseq = kseq[1:] + kseq[:1]
        rots = rots[1:] + rots[:1]
    return x0, x1v


def _random_bits_np(k, n, partitionable):
    if partitionable:
        b1, b2 = _threefry2x32_np(
            k[0], k[1], np.zeros(n, np.uint32), np.arange(n, dtype=np.uint32))
        return b1 ^ b2
    cnt = np.arange(n, dtype=np.uint32)
    if n % 2:
        cnt = np.append(cnt, np.uint32(0))
    h = len(cnt) // 2
    b1, b2 = _threefry2x32_np(k[0], k[1], cnt[:h], cnt[h:])
    return np.concatenate([b1, b2])[:n]


def _split_np(k1, k2, partitionable):
    if partitionable:
        b1, b2 = _threefry2x32_np(
            k1, k2, np.zeros(2, np.uint32), np.arange(2, dtype=np.uint32))
        return [(b1[0], b2[0]), (b1[1], b2[1])]
    b1, b2 = _threefry2x32_np(
        k1, k2, np.array([0, 1], np.uint32), np.array([2, 3], np.uint32))
    o = np.concatenate([b1, b2])
    return [(o[0], o[1]), (o[2], o[3])]


@functools.lru_cache(maxsize=None)
def _comp_ind_np(batch_size: int, num_samples: int, num_comp: int):
    partitionable = bool(jax.config.jax_threefry_partitionable)
    n = batch_size * num_samples
    khi, klo = _split_np(np.uint32(0), np.uint32(42), partitionable)
    hi_bits = _random_bits_np(khi, n, partitionable)
    lo_bits = _random_bits_np(klo, n, partitionable)
    span = np.uint32(num_comp)
    mult = np.uint32(((2 ** 16 % num_comp) ** 2 % 2 ** 32) % num_comp)
    off = ((hi_bits % span) * mult + (lo_bits % span)).astype(np.uint32) % span
    return off.astype(np.int32)


@functools.lru_cache(maxsize=None)
def _build_sc_kernel(batch: int, nsamp: int, d: int, num_comp: int):
    info = plsc.get_sparse_core_info()
    nc, ns = info.num_cores, info.num_subcores
    nw = nc * ns
    assert d % nw == 0 and batch % _LANES == 0 and num_comp % 8 == 0
    ch_pw = d // nw

    mesh = plsc.VectorSubcoreMesh(core_axis_name="c", subcore_axis_name="s")

    @functools.partial(
        pl.kernel,
        out_type=jax.ShapeDtypeStruct((nsamp, d, batch), jnp.float32),
        mesh=mesh,
        compiler_params=pltpu.CompilerParams(
            use_tc_tiling_on_sc=True, needs_layout_passes=False),
        scratch_types=[
            pltpu.VMEM((num_comp + (-num_comp) % 128,), jnp.float32),
            pltpu.VMEM((2, batch), jnp.int32),
            pltpu.VMEM((2, batch), jnp.float32),
            pltpu.SemaphoreType.DMA((2,)),
            pltpu.SemaphoreType.DMA((2,)),
            pltpu.SemaphoreType.DMA((2,)),
        ],
    )
    def gmm(idx_hbm, noise_hbm, means_hbm, out_hbm,
            table_v, idx_v, noise_v, idx_sem, noi_sem, out_sem):
        wid = lax.axis_index("s") * nc + lax.axis_index("c")

        def idx_copy(s, sl):
            return pltpu.make_async_copy(
                idx_hbm.at[pl.ds(s * batch, batch)], idx_v.at[sl], idx_sem.at[sl])

        def noi_copy(s, c, sl):
            return pltpu.make_async_copy(
                noise_hbm.at[s, c], noise_v.at[sl], noi_sem.at[sl])

        def out_copy(s, c, sl):
            return pltpu.make_async_copy(
                noise_v.at[sl], out_hbm.at[s, c], out_sem.at[sl])

        for ci in range(ch_pw):
            c = wid + nw * ci
            pltpu.sync_copy(means_hbm.at[c], table_v)
            idx_copy(0, 0).start()
            noi_copy(0, c, 0).start()

            def sample(s, carry):
                sl = lax.rem(s, 2)
                s1 = lax.rem(s + 1, 2)

                @pl.when(s + 1 < nsamp)
                def _():
                    @pl.when(s >= 1)
                    def _():
                        out_copy(s - 1, c, s1).wait()
                    idx_copy(s + 1, s1).start()
                    noi_copy(s + 1, c, s1).start()

                idx_copy(s, sl).wait()
                noi_copy(s, c, sl).wait()

                @plsc.parallel_loop(0, batch, step=_LANES, unroll=8)
                def _(i):
                    v = pl.ds(i, _LANES)
                    iv = idx_v[sl, v]
                    g = plsc.load_gather(table_v, [iv])
                    noise_v[sl, v] = noise_v[sl, v] * 0.1 + g
                out_copy(s, c, sl).start()
                return carry

            lax.fori_loop(0, nsamp, sample, 0)
            out_copy(nsamp - 1, c, (nsamp - 1) % 2).wait()
            out_copy(nsamp - 2, c, (nsamp - 2) % 2).wait()

    return gmm


def kernel(input, noise, target_size, means):
    del input, target_size
    b, s, d = noise.shape
    num_comp = means.shape[0]
    idx_np = _comp_ind_np(b, s, num_comp)
    idx_t = jnp.asarray(np.ascontiguousarray(idx_np.reshape(b, s).T).reshape(-1))
    noise_t = jnp.transpose(noise, (1, 2, 0))
    means_t = jnp.pad(jnp.transpose(means, (1, 0)),
                      ((0, 0), (0, (-num_comp) % 128)))
    out_t = _build_sc_kernel(b, s, d, num_comp)(idx_t, noise_t, means_t)
    return jnp.transpose(out_t, (2, 0, 1))


# speedup vs baseline: 1.7306x; 1.1269x over previous
import functools

import jax
import jax.numpy as jnp
import numpy as np
from jax import lax
from jax.experimental import pallas as pl
from jax.experimental.pallas import tpu as pltpu
from jax.experimental.pallas import tpu_sc as plsc

_LANES = 16


def _threefry2x32_np(k1, k2, x1, x2):
    rot = [(13, 15, 26, 6), (17, 29, 16, 24)]
    ks = [np.uint32(k1), np.uint32(k2),
          np.uint32(np.uint32(k1) ^ np.uint32(k2) ^ np.uint32(0x1BD11BDA))]
    x0 = (x1 + ks[0]).astype(np.uint32)
    x1v = (x2 + ks[1]).astype(np.uint32)
    kseq = [ks[1], ks[2], ks[0]]
    rots = [rot[0], rot[1]]
    for i in range(5):
        for r in rots[0]:
            x0 = (x0 + x1v).astype(np.uint32)
            x1v = ((x1v << np.uint32(r)) | (x1v >> np.uint32(32 - r))).astype(np.uint32)
            x1v = x0 ^ x1v
        x0 = (x0 + kseq[0]).astype(np.uint32)
        x1v = (x1v + kseq[1] + np.uint32(i + 1)).astype(np.uint32)
        kseq = kseq[1:] + kseq[:1]
        rots = rots[1:] + rots[:1]
    return x0, x1v


def _random_bits_np(k, n, partitionable):
    if partitionable:
        b1, b2 = _threefry2x32_np(
            k[0], k[1], np.zeros(n, np.uint32), np.arange(n, dtype=np.uint32))
        return b1 ^ b2
    cnt = np.arange(n, dtype=np.uint32)
    if n % 2:
        cnt = np.append(cnt, np.uint32(0))
    h = len(cnt) // 2
    b1, b2 = _threefry2x32_np(k[0], k[1], cnt[:h], cnt[h:])
    return np.concatenate([b1, b2])[:n]


def _split_np(k1, k2, partitionable):
    if partitionable:
        b1, b2 = _threefry2x32_np(
            k1, k2, np.zeros(2, np.uint32), np.arange(2, dtype=np.uint32))
        return [(b1[0], b2[0]), (b1[1], b2[1])]
    b1, b2 = _threefry2x32_np(
        k1, k2, np.array([0, 1], np.uint32), np.array([2, 3], np.uint32))
    o = np.concatenate([b1, b2])
    return [(o[0], o[1]), (o[2], o[3])]


@functools.lru_cache(maxsize=None)
def _comp_ind_np(batch_size: int, num_samples: int, num_comp: int):
    partitionable = bool(jax.config.jax_threefry_partitionable)
    n = batch_size * num_samples
    khi, klo = _split_np(np.uint32(0), np.uint32(42), partitionable)
    hi_bits = _random_bits_np(khi, n, partitionable)
    lo_bits = _random_bits_np(klo, n, partitionable)
    span = np.uint32(num_comp)
    mult = np.uint32(((2 ** 16 % num_comp) ** 2 % 2 ** 32) % num_comp)
    off = ((hi_bits % span) * mult + (lo_bits % span)).astype(np.uint32) % span
    return off.astype(np.int32)


@functools.lru_cache(maxsize=None)
def _build_sc_kernel(batch: int, nsamp: int, d: int, num_comp: int):
    info = plsc.get_sparse_core_info()
    nc, ns = info.num_cores, info.num_subcores
    nw = nc * ns
    assert d % nw == 0 and batch % _LANES == 0 and num_comp % 8 == 0
    ch_pw = d // nw

    mesh = plsc.VectorSubcoreMesh(core_axis_name="c", subcore_axis_name="s")

    @functools.partial(
        pl.kernel,
        out_type=jax.ShapeDtypeStruct((nsamp, d, batch), jnp.float32),
        mesh=mesh,
        compiler_params=pltpu.CompilerParams(
            use_tc_tiling_on_sc=True, needs_layout_passes=False),
        scratch_types=[
            pltpu.VMEM((num_comp,), jnp.float32),
            pltpu.VMEM((2, batch), jnp.int32),
            pltpu.VMEM((2, batch), jnp.float32),
            pltpu.SemaphoreType.DMA((2,)),
            pltpu.SemaphoreType.DMA((2,)),
            pltpu.SemaphoreType.DMA((2,)),
        ],
    )
    def gmm(idx_hbm, noise_hbm, means_hbm, out_hbm,
            table_v, idx_v, noise_v, idx_sem, noi_sem, out_sem):
        wid = lax.axis_index("s") * nc + lax.axis_index("c")

        def idx_copy(s, sl):
            return pltpu.make_async_copy(
                idx_hbm.at[pl.ds(s * batch, batch)], idx_v.at[sl], idx_sem.at[sl])

        def noi_copy(s, c, sl):
            return pltpu.make_async_copy(
                noise_hbm.at[s, c], noise_v.at[sl], noi_sem.at[sl])

        def out_copy(s, c, sl):
            return pltpu.make_async_copy(
                noise_v.at[sl], out_hbm.at[s, c], out_sem.at[sl])

        for ci in range(ch_pw):
            c = wid + nw * ci
            pltpu.sync_copy(means_hbm.at[c], table_v)
            idx_copy(0, 0).start()
            noi_copy(0, c, 0).start()

            def sample(s, carry):
                sl = lax.rem(s, 2)
                s1 = lax.rem(s + 1, 2)

                @pl.when(s + 1 < nsamp)
                def _():
                    @pl.when(s >= 1)
                    def _():
                        out_copy(s - 1, c, s1).wait()
                    idx_copy(s + 1, s1).start()
                    noi_copy(s + 1, c, s1).start()

                idx_copy(s, sl).wait()
                noi_copy(s, c, sl).wait()

                @plsc.parallel_loop(0, batch, step=_LANES, unroll=8)
                def _(i):
                    v = pl.ds(i, _LANES)
                    iv = idx_v[sl, v]
                    g = plsc.load_gather(table_v, [iv])
                    noise_v[sl, v] = noise_v[sl, v] * 0.1 + g
                out_copy(s, c, sl).start()
                return carry

            lax.fori_loop(0, nsamp, sample, 0)
            out_copy(nsamp - 1, c, (nsamp - 1) % 2).wait()
            out_copy(nsamp - 2, c, (nsamp - 2) % 2).wait()

    return gmm


def kernel(input, noise, target_size, means):
    del input, target_size
    b, s, d = noise.shape
    num_comp = means.shape[0]
    idx_np = _comp_ind_np(b, s, num_comp)
    idx_t = jnp.asarray(np.ascontiguousarray(idx_np.reshape(b, s).T).reshape(-1))
    noise_t = jnp.transpose(noise, (1, 2, 0))
    means_t = jnp.transpose(means, (1, 0))
    out_t = _build_sc_kernel(b, s, d, num_comp)(idx_t, noise_t, means_t)
    return jnp.transpose(out_t, (2, 0, 1))
